# baseline (speedup 1.0000x reference)
import jax
import jax.numpy as jnp
from jax.experimental import pallas as pl
from jax.experimental.pallas import tpu as pltpu

_N = 65536
_D = 64
_K = 256
_B = 512
_P = 128

_BIG_I32 = 2**31 - 1

_CB = 128
_DCB = 256


def _fps_kernel(f_ref, att_ref, out_ref, fn_ref, base_ref, md_ref, idx_ref):
    def norm_chunk(c, carry):
        sl = pl.ds(c * _CB, _CB)
        f = f_ref[:, sl, :]
        n2 = jnp.sum(f * f, axis=0, keepdims=True)
        n = jnp.sqrt(n2)
        fn_ref[:, sl, :] = f / jnp.maximum(n, 1e-12)
        return carry

    jax.lax.fori_loop(0, _B // _CB, norm_chunk, 0)

    att = att_ref[...]
    a_min = jnp.min(att)
    a_max = jnp.max(att)
    an = (att - a_min) / (a_max - a_min + 1e-10)

    row_ids = jax.lax.broadcasted_iota(jnp.int32, (_B, _P), 0)
    col_ids = jax.lax.broadcasted_iota(jnp.int32, (_B, _P), 1)
    idx = row_ids * _P + col_ids
    idx_ref[...] = idx

    m0 = jnp.max(att)
    first = jnp.min(jnp.where(att == m0, idx, _BIG_I32))
    out_ref[0] = first

    base_ref[...] = jnp.where(idx == first, -jnp.inf, 0.5 * an)
    md_ref[...] = jnp.full((_B, _P), jnp.inf, dtype=jnp.float32)

    lane3 = jax.lax.broadcasted_iota(jnp.int32, (1, 1, _P), 2)
    lane2 = jax.lax.broadcasted_iota(jnp.int32, (1, _P), 1)

    def step(i, carry):
        last = out_ref[i - 1]
        b0 = last // _P
        p0 = last % _P
        slab = fn_ref[:, pl.ds(b0, 1), :]
        onehot = (lane3 == p0).astype(jnp.float32)
        row = jnp.sum(slab * onehot, axis=2, keepdims=True)

        def dist_chunk(c, carry):
            m, bi = carry
            sl = pl.ds(c * _DCB, _DCB)
            diff = fn_ref[:, sl, :] - row
            d2 = jnp.sum(diff * diff, axis=0)
            dist = jnp.sqrt(d2)
            md = jnp.minimum(md_ref[sl, :], dist)
            md_ref[sl, :] = md
            cb = base_ref[sl, :] + 0.5 * md
            mc = jnp.max(cb)
            ic = jnp.min(jnp.where(cb == mc, idx_ref[sl, :], _BIG_I32))
            take = mc > m
            return (jnp.where(take, mc, m), jnp.where(take, ic, bi))

        _, best = jax.lax.fori_loop(
            0, _B // _DCB, dist_chunk,
            (jnp.float32(-jnp.inf), jnp.int32(0)))

        out_ref[i] = best
        bb = best // _P
        pp = best % _P
        brow = base_ref[pl.ds(bb, 1), :]
        base_ref[pl.ds(bb, 1), :] = jnp.where(lane2 == pp, -jnp.inf, brow)
        return carry

    jax.lax.fori_loop(1, _K, step, 0)


def kernel(features, attention_scores, k):
    del k
    fT = features.reshape(_B, _P, _D).transpose(2, 0, 1)
    att = attention_scores.reshape(_B, _P)

    out = pl.pallas_call(
        _fps_kernel,
        out_shape=jax.ShapeDtypeStruct((_K,), jnp.int32),
        in_specs=[
            pl.BlockSpec(memory_space=pltpu.MemorySpace.VMEM),
            pl.BlockSpec(memory_space=pltpu.MemorySpace.VMEM),
        ],
        out_specs=pl.BlockSpec(memory_space=pltpu.MemorySpace.SMEM),
        scratch_shapes=[
            pltpu.VMEM((_D, _B, _P), jnp.float32),
            pltpu.VMEM((_B, _P), jnp.float32),
            pltpu.VMEM((_B, _P), jnp.float32),
            pltpu.VMEM((_B, _P), jnp.int32),
        ],
        compiler_params=pltpu.CompilerParams(
            vmem_limit_bytes=100 * 1024 * 1024,
        ),
    )(fT, att)
    return out


# speedup vs baseline: 1.5238x; 1.5238x over previous
import jax
import jax.numpy as jnp
from jax.experimental import pallas as pl
from jax.experimental.pallas import tpu as pltpu

_N = 65536
_D = 64
_K = 256
_B = 512
_P = 128

_BIG_I32 = 2**31 - 1

_CB = 128
_DCB = 256


def _fps_kernel(f_ref, att_ref, out_ref, fn_ref, base_ref, md_ref, idx_ref):
    def norm_chunk(c, carry):
        sl = pl.ds(c * _CB, _CB)
        f = f_ref[:, sl, :]
        n2 = jnp.sum(f * f, axis=0, keepdims=True)
        n = jnp.sqrt(n2)
        fn_ref[:, sl, :] = f / jnp.maximum(n, 1e-12)
        return carry

    jax.lax.fori_loop(0, _B // _CB, norm_chunk, 0)

    att = att_ref[...]
    a_min = jnp.min(att)
    a_max = jnp.max(att)
    an = (att - a_min) / (a_max - a_min + 1e-10)

    row_ids = jax.lax.broadcasted_iota(jnp.int32, (_B, _P), 0)
    col_ids = jax.lax.broadcasted_iota(jnp.int32, (_B, _P), 1)
    idx = row_ids * _P + col_ids
    idx_ref[...] = idx

    m0 = jnp.max(att)
    first = jnp.min(jnp.where(att == m0, idx, _BIG_I32))
    out_ref[0] = first

    base_ref[...] = jnp.where(idx == first, -jnp.inf, 0.5 * an)
    md_ref[...] = jnp.full((_B, _P), jnp.inf, dtype=jnp.float32)

    lane3 = jax.lax.broadcasted_iota(jnp.int32, (1, 1, _P), 2)
    lane2 = jax.lax.broadcasted_iota(jnp.int32, (1, _P), 1)

    def step(i, carry):
        last = out_ref[i - 1]
        b0 = last // _P
        p0 = last % _P
        slab = fn_ref[:, pl.ds(b0, 1), :]
        onehot = (lane3 == p0).astype(jnp.float32)
        row = jnp.sum(slab * onehot, axis=2, keepdims=True)

        def dist_chunk(c, carry):
            sl = pl.ds(c * _DCB, _DCB)
            diff = fn_ref[:, sl, :] - row
            d2 = jnp.sum(diff * diff, axis=0)
            dist = jnp.sqrt(d2)
            md_ref[sl, :] = jnp.minimum(md_ref[sl, :], dist)
            return carry

        jax.lax.fori_loop(0, _B // _DCB, dist_chunk, 0)

        comb = base_ref[...] + 0.5 * md_ref[...]
        m = jnp.max(comb)
        best = jnp.min(jnp.where(comb == m, idx_ref[...], _BIG_I32))
        out_ref[i] = best
        bb = best // _P
        pp = best % _P
        brow = base_ref[pl.ds(bb, 1), :]
        base_ref[pl.ds(bb, 1), :] = jnp.where(lane2 == pp, -jnp.inf, brow)
        return carry

    jax.lax.fori_loop(1, _K, step, 0)


def kernel(features, attention_scores, k):
    del k
    fT = features.reshape(_B, _P, _D).transpose(2, 0, 1)
    att = attention_scores.reshape(_B, _P)

    out = pl.pallas_call(
        _fps_kernel,
        out_shape=jax.ShapeDtypeStruct((_K,), jnp.int32),
        in_specs=[
            pl.BlockSpec(memory_space=pltpu.MemorySpace.VMEM),
            pl.BlockSpec(memory_space=pltpu.MemorySpace.VMEM),
        ],
        out_specs=pl.BlockSpec(memory_space=pltpu.MemorySpace.SMEM),
        scratch_shapes=[
            pltpu.VMEM((_D, _B, _P), jnp.float32),
            pltpu.VMEM((_B, _P), jnp.float32),
            pltpu.VMEM((_B, _P), jnp.float32),
            pltpu.VMEM((_B, _P), jnp.int32),
        ],
        compiler_params=pltpu.CompilerParams(
            vmem_limit_bytes=100 * 1024 * 1024,
        ),
    )(fT, att)
    return out


# speedup vs baseline: 1.5241x; 1.0002x over previous
import jax
import jax.numpy as jnp
from jax.experimental import pallas as pl
from jax.experimental.pallas import tpu as pltpu

_N = 65536
_D = 64
_K = 256
_B = 512
_P = 128

_BIG_I32 = 2**31 - 1

_CB = 128
_DCB = 256


def _fps_kernel(f_ref, att_ref, out_ref, fn_ref, base_ref, md_ref, idx_ref):
    def norm_chunk(c, carry):
        sl = pl.ds(c * _CB, _CB)
        f = f_ref[:, sl, :]
        n2 = jnp.sum(f * f, axis=0, keepdims=True)
        n = jnp.sqrt(n2)
        fn_ref[:, sl, :] = f / jnp.maximum(n, 1e-12)
        return carry

    jax.lax.fori_loop(0, _B // _CB, norm_chunk, 0)

    att = att_ref[...]
    a_min = jnp.min(att)
    a_max = jnp.max(att)
    an = (att - a_min) / (a_max - a_min + 1e-10)

    row_ids = jax.lax.broadcasted_iota(jnp.int32, (_B, _P), 0)
    col_ids = jax.lax.broadcasted_iota(jnp.int32, (_B, _P), 1)
    idx = row_ids * _P + col_ids
    idx_ref[...] = idx

    m0 = jnp.max(att)
    first = jnp.min(jnp.where(att == m0, idx, _BIG_I32))
    out_ref[0] = first

    base_ref[...] = jnp.where(idx == first, -jnp.inf, 0.5 * an)
    md_ref[...] = jnp.full((_B, _P), jnp.inf, dtype=jnp.float32)

    lane3 = jax.lax.broadcasted_iota(jnp.int32, (1, 1, _P), 2)
    lane2 = jax.lax.broadcasted_iota(jnp.int32, (1, _P), 1)

    def step(i, carry):
        last = out_ref[i - 1]
        b0 = last // _P
        p0 = last % _P
        slab = fn_ref[:, pl.ds(b0, 1), :]
        onehot = (lane3 == p0).astype(jnp.float32)
        row = jnp.sum(slab * onehot, axis=2, keepdims=True)

        def dist_chunk(c, carry):
            sl = pl.ds(c * _DCB, _DCB)
            diff = fn_ref[:, sl, :] - row
            d2 = jnp.sum(diff * diff, axis=0)
            dist = jnp.sqrt(d2)
            md_ref[sl, :] = jnp.minimum(md_ref[sl, :], dist)
            return carry

        jax.lax.fori_loop(0, _B // _DCB, dist_chunk, 0)

        comb = base_ref[...] + 0.5 * md_ref[...]
        m = jnp.max(comb)
        best = jnp.min(jnp.where(comb == m, idx_ref[...], _BIG_I32))
        out_ref[i] = best
        bb = best // _P
        pp = best % _P
        brow = base_ref[pl.ds(bb, 1), :]
        base_ref[pl.ds(bb, 1), :] = jnp.where(lane2 == pp, -jnp.inf, brow)
        return carry

    jax.lax.fori_loop(1, _K, step, 0, unroll=2)


def kernel(features, attention_scores, k):
    del k
    fT = features.reshape(_B, _P, _D).transpose(2, 0, 1)
    att = attention_scores.reshape(_B, _P)

    out = pl.pallas_call(
        _fps_kernel,
        out_shape=jax.ShapeDtypeStruct((_K,), jnp.int32),
        in_specs=[
            pl.BlockSpec(memory_space=pltpu.MemorySpace.VMEM),
            pl.BlockSpec(memory_space=pltpu.MemorySpace.VMEM),
        ],
        out_specs=pl.BlockSpec(memory_space=pltpu.MemorySpace.SMEM),
        scratch_shapes=[
            pltpu.VMEM((_D, _B, _P), jnp.float32),
            pltpu.VMEM((_B, _P), jnp.float32),
            pltpu.VMEM((_B, _P), jnp.float32),
            pltpu.VMEM((_B, _P), jnp.int32),
        ],
        compiler_params=pltpu.CompilerParams(
            vmem_limit_bytes=100 * 1024 * 1024,
        ),
    )(fT, att)
    return out
